# baseline (speedup 1.0000x reference)
import jax
import jax.numpy as jnp
from jax import lax
from jax.experimental import pallas as pl
from jax.experimental.pallas import tpu as pltpu
from jax.experimental.pallas import tpu_sc as plsc

N_NODES = 100000
N_EDGES = 3200000
DN = 20
DE = 16
NP = 24
DH = 10
DHP = 16

NC = 2
NS = 16
NW = NC * NS

_CHUNK = 2000
_EPW = N_EDGES // NW
_ITERS = _EPW // _CHUNK

_mesh = plsc.VectorSubcoreMesh(
    core_axis_name="c", subcore_axis_name="s", num_cores=NC, num_subcores=NS)

_sc_params = pltpu.CompilerParams(use_tc_tiling_on_sc=False)


def _sc_gather_body(ptab, qtab, senders, receivers, z,
                    idx_s, idx_r, rows_v, sem, sem2):
    c = lax.axis_index("c")
    s = lax.axis_index("s")
    wid = s * NC + c
    base0 = wid * _EPW

    def body(i, carry):
        base = base0 + i * _CHUNK
        pltpu.sync_copy(senders.at[pl.ds(base, _CHUNK)], idx_s)
        pltpu.sync_copy(receivers.at[pl.ds(base, _CHUNK)], idx_r)
        pltpu.async_copy(ptab.at[idx_s], rows_v, sem).wait()
        pltpu.async_copy(qtab.at[idx_r], rows_v, sem2, add=True).wait()
        pltpu.sync_copy(rows_v, z.at[pl.ds(base, _CHUNK)])
        return carry

    lax.fori_loop(0, _ITERS, body, 0, unroll=False)


_sc_gather = pl.kernel(
    _sc_gather_body,
    out_type=jax.ShapeDtypeStruct((N_EDGES, 64), jnp.bfloat16),
    mesh=_mesh,
    scratch_types=[
        pltpu.VMEM((_CHUNK,), jnp.int32),
        pltpu.VMEM((_CHUNK,), jnp.int32),
        pltpu.VMEM((_CHUNK, 64), jnp.bfloat16),
        pltpu.SemaphoreType.DMA,
        pltpu.SemaphoreType.DMA,
    ],
    compiler_params=_sc_params,
)


_SCHUNK = 800
_EPT = N_EDGES // NS
_SITERS = _EPT // _SCHUNK
_NVR = _SCHUNK // 16


def _sc_scatter_body(msgs, receivers, zeros, agg2, idx_v, ridx_v, msg_v, sem,
                     accum):
    c = lax.axis_index("c")
    s = lax.axis_index("s")
    stripe = N_NODES // NS
    t0 = s * stripe
    pltpu.sync_copy(zeros.at[pl.ds(t0, stripe)], accum.at[pl.ds(t0, stripe)])

    base0 = c * N_EDGES + s * _EPT

    def init(j, carry):
        idx_v[pl.ds(j * 16, 16)] = base0 + j * 16 + lax.iota(jnp.int32, 16)
        return carry

    lax.fori_loop(0, _NVR, init, 0, unroll=False)
    plsc.subcore_barrier()

    def body(i, carry):
        base = base0 + i * _SCHUNK
        pltpu.sync_copy(receivers.at[pl.ds(base - c * N_EDGES, _SCHUNK)],
                        ridx_v)
        pltpu.async_copy(msgs.at[idx_v], msg_v, sem).wait()
        pltpu.sync_copy(msg_v, accum.at[ridx_v], add=True)

        def bump(j, carry2):
            sl = pl.ds(j * 16, 16)
            idx_v[sl] = idx_v[sl] + _SCHUNK
            return carry2

        lax.fori_loop(0, _NVR, bump, 0, unroll=False)
        return carry

    lax.fori_loop(0, _SITERS, body, 0, unroll=False)
    plsc.subcore_barrier()
    pltpu.sync_copy(accum.at[pl.ds(t0, stripe)], agg2.at[c, pl.ds(t0, stripe)])


_sc_scatter = pl.kernel(
    _sc_scatter_body,
    out_type=jax.ShapeDtypeStruct((NC, N_NODES, DHP), jnp.float32),
    mesh=_mesh,
    scratch_types=[
        pltpu.VMEM((_SCHUNK,), jnp.int32),
        pltpu.VMEM((_SCHUNK,), jnp.int32),
        pltpu.VMEM((_SCHUNK, DHP), jnp.float32),
        pltpu.SemaphoreType.DMA,
        pltpu.VMEM_SHARED((N_NODES, DHP), jnp.float32),
    ],
    compiler_params=_sc_params,
)


_BNP = 2000


def _proj_body(nodes, ws, wr, p, q):
    n = nodes[...]
    p[...] = jnp.dot(n, ws[...],
                     preferred_element_type=jnp.float32).astype(jnp.bfloat16)
    q[...] = jnp.dot(n, wr[...],
                     preferred_element_type=jnp.float32).astype(jnp.bfloat16)


def _full2d(shape):
    return pl.BlockSpec(shape, lambda i: (0, 0))


_proj = pl.pallas_call(
    _proj_body,
    grid=(N_NODES // _BNP,),
    in_specs=[
        pl.BlockSpec((_BNP, DN), lambda i: (i, 0)),
        _full2d((DN, 64)),
        _full2d((DN, 64)),
    ],
    out_specs=[pl.BlockSpec((_BNP, 64), lambda i: (i, 0)),
               pl.BlockSpec((_BNP, 64), lambda i: (i, 0))],
    out_shape=[jax.ShapeDtypeStruct((N_NODES, 64), jnp.bfloat16),
               jax.ShapeDtypeStruct((N_NODES, 64), jnp.bfloat16)],
)


_BE = 6400
_BE8 = _BE // 8


def _edge_mlp_body(ef8, z8, w1, b1, w2, b2, w3, b3, out):
    h = jnp.maximum(
        jnp.dot(ef8[...], w1[...], preferred_element_type=jnp.float32)
        + z8[...].astype(jnp.float32) + b1[...], 0.0)
    h = jnp.maximum(
        jnp.dot(h, w2[...], preferred_element_type=jnp.float32) + b2[...], 0.0)
    y = jnp.dot(h, w3[...], preferred_element_type=jnp.float32) + b3[...]
    out[0] = y[:, :128]
    out[1] = y[:, 128:]


_edge_mlp = pl.pallas_call(
    _edge_mlp_body,
    grid=(N_EDGES // _BE,),
    in_specs=[
        pl.BlockSpec((_BE8, 128), lambda i: (i, 0)),
        pl.BlockSpec((_BE8, 512), lambda i: (i, 0)),
        _full2d((128, 512)),
        _full2d((1, 512)),
        _full2d((512, 256)),
        _full2d((1, 256)),
        _full2d((256, 256)),
        _full2d((1, 256)),
    ],
    out_specs=pl.BlockSpec((NC, _BE8, 128), lambda i: (0, i, 0)),
    out_shape=jax.ShapeDtypeStruct((NC, N_EDGES // 8, 128), jnp.float32),
)


_BN8 = 1600
_N8P = 12800


def _node_mlp_body(nodes8, agg8, w1, b1, wa, wb, w2, b2, w3, b3, out):
    h = (jnp.dot(nodes8[...], w1[...], preferred_element_type=jnp.float32)
         + jnp.dot(agg8[0], wa[...], preferred_element_type=jnp.float32)
         + jnp.dot(agg8[1], wb[...], preferred_element_type=jnp.float32)
         + b1[...])
    h = jnp.maximum(h, 0.0)
    h = jnp.maximum(
        jnp.dot(h, w2[...], preferred_element_type=jnp.float32) + b2[...], 0.0)
    out[...] = jnp.dot(h, w3[...], preferred_element_type=jnp.float32) + b3[...]


_node_mlp = pl.pallas_call(
    _node_mlp_body,
    grid=(_N8P // _BN8,),
    in_specs=[
        pl.BlockSpec((_BN8, 160), lambda i: (i, 0)),
        pl.BlockSpec((NC, _BN8, 128), lambda i: (0, i, 0)),
        _full2d((160, 512)),
        _full2d((1, 512)),
        _full2d((128, 512)),
        _full2d((128, 512)),
        _full2d((512, 256)),
        _full2d((1, 256)),
        _full2d((256, 160)),
        _full2d((1, 160)),
    ],
    out_specs=pl.BlockSpec((_BN8, 160), lambda i: (i, 0)),
    out_shape=jax.ShapeDtypeStruct((_N8P, 160), jnp.float32),
)


def _blockdiag(w, r):
    k, n = w.shape
    out = jnp.zeros((r * k, r * n), w.dtype)
    for i in range(r):
        out = lax.dynamic_update_slice(out, w, (i * k, i * n))
    return out


def kernel(nodes, edge_features, senders, receivers,
           ew1, eb1, ew2, eb2, ew3, eb3,
           nw1, nb1, nw2, nb2, nw3, nb3):
    p, q = _proj(nodes, ew1[DE:DE + DN], ew1[DE + DN:])
    z = _sc_gather(p, q, senders, receivers)
    ef8 = edge_features.reshape(N_EDGES // 8, 8 * DE)
    z8 = z.reshape(N_EDGES // 8, 8 * 64)
    w1e8 = _blockdiag(ew1[:DE], 8)
    b1_8 = (jnp.tile(eb1, 8)[None, :] + 0.0)
    w2_8 = _blockdiag(ew2, 8)
    b2_8 = jnp.tile(eb2, 8)[None, :]
    w3a = jnp.pad(ew3[:, :DH], ((0, 0), (0, DHP - DH)))
    w3b = jnp.pad(ew3[:, DH:], ((0, 0), (0, DHP - DH)))
    w3_8 = jnp.concatenate([_blockdiag(w3a, 8), _blockdiag(w3b, 8)], axis=1)
    b3a = jnp.pad(eb3[:DH], (0, DHP - DH))
    b3b = jnp.pad(eb3[DH:], (0, DHP - DH))
    b3_8 = jnp.concatenate([jnp.tile(b3a, 8), jnp.tile(b3b, 8)])[None, :]
    msgs = _edge_mlp(ef8, z8, w1e8, b1_8, w2_8, b2_8, w3_8, b3_8)
    agg2 = _sc_scatter(msgs.reshape(NC * N_EDGES, DHP), receivers,
                       jnp.zeros((N_NODES, DHP), jnp.float32))
    nodes8 = jnp.pad(nodes.reshape(N_NODES // 8, 8 * DN),
                     ((0, _N8P - N_NODES // 8), (0, 0)))
    agg8 = jnp.pad(agg2.reshape(NC, N_NODES // 8, 8 * DHP),
                   ((0, 0), (0, _N8P - N_NODES // 8), (0, 0)))
    w1n8 = _blockdiag(nw1[:DN], 8)
    w1a8 = _blockdiag(jnp.pad(nw1[DN:DN + DH], ((0, DHP - DH), (0, 0))), 8)
    w1b8 = _blockdiag(jnp.pad(nw1[DN + DH:], ((0, DHP - DH), (0, 0))), 8)
    nb1_8 = jnp.tile(nb1, 8)[None, :]
    w2n8 = _blockdiag(nw2, 8)
    nb2_8 = jnp.tile(nb2, 8)[None, :]
    w3n8 = _blockdiag(nw3, 8)
    nb3_8 = jnp.tile(nb3, 8)[None, :]
    out8 = _node_mlp(nodes8, agg8, w1n8, nb1_8, w1a8, w1b8,
                     w2n8, nb2_8, w3n8, nb3_8)
    return out8[:N_NODES // 8].reshape(N_NODES, DN)


# speedup vs baseline: 1.1779x; 1.1779x over previous
import jax
import jax.numpy as jnp
from jax import lax
from jax.experimental import pallas as pl
from jax.experimental.pallas import tpu as pltpu
from jax.experimental.pallas import tpu_sc as plsc

N_NODES = 100000
N_EDGES = 3200000
DN = 20
DE = 16
NP = 24
DH = 10
DHP = 16

NC = 2
NS = 16
NW = NC * NS

_CHUNK = 1000
_EPW = N_EDGES // NW
_ITERS = _EPW // _CHUNK

_mesh = plsc.VectorSubcoreMesh(
    core_axis_name="c", subcore_axis_name="s", num_cores=NC, num_subcores=NS)

_sc_params = pltpu.CompilerParams(use_tc_tiling_on_sc=False)


def _sc_gather_body(ptab, qtab, senders, receivers, z,
                    idx_s, idx_r, rows_v, sem, sem2):
    c = lax.axis_index("c")
    s = lax.axis_index("s")
    wid = s * NC + c
    base0 = wid * _EPW

    def body(i, carry):
        base = base0 + i * _CHUNK
        pltpu.sync_copy(senders.at[pl.ds(base, _CHUNK)], idx_s)
        pltpu.sync_copy(receivers.at[pl.ds(base, _CHUNK)], idx_r)
        pltpu.async_copy(ptab.at[idx_s], rows_v, sem).wait()
        pltpu.async_copy(qtab.at[idx_r], rows_v, sem2, add=True).wait()
        pltpu.sync_copy(rows_v, z.at[pl.ds(base, _CHUNK)])
        return carry

    lax.fori_loop(0, _ITERS, body, 0, unroll=False)


_sc_gather = pl.kernel(
    _sc_gather_body,
    out_type=jax.ShapeDtypeStruct((N_EDGES, 64), jnp.float32),
    mesh=_mesh,
    scratch_types=[
        pltpu.VMEM((_CHUNK,), jnp.int32),
        pltpu.VMEM((_CHUNK,), jnp.int32),
        pltpu.VMEM((_CHUNK, 64), jnp.float32),
        pltpu.SemaphoreType.DMA,
        pltpu.SemaphoreType.DMA,
    ],
    compiler_params=_sc_params,
)


_SCHUNK = 800
_EPT = N_EDGES // NS
_SITERS = _EPT // _SCHUNK
_NVR = _SCHUNK // 16


def _sc_scatter_body(msgs, receivers, zeros, agg2, idx_v, ridx_v, msg_v, sem,
                     accum):
    c = lax.axis_index("c")
    s = lax.axis_index("s")
    stripe = N_NODES // NS
    t0 = s * stripe
    pltpu.sync_copy(zeros.at[pl.ds(t0, stripe)], accum.at[pl.ds(t0, stripe)])

    base0 = c * N_EDGES + s * _EPT

    def init(j, carry):
        idx_v[pl.ds(j * 16, 16)] = base0 + j * 16 + lax.iota(jnp.int32, 16)
        return carry

    lax.fori_loop(0, _NVR, init, 0, unroll=False)
    plsc.subcore_barrier()

    def body(i, carry):
        base = base0 + i * _SCHUNK
        pltpu.sync_copy(receivers.at[pl.ds(base - c * N_EDGES, _SCHUNK)],
                        ridx_v)
        pltpu.async_copy(msgs.at[idx_v], msg_v, sem).wait()
        pltpu.sync_copy(msg_v, accum.at[ridx_v], add=True)

        def bump(j, carry2):
            sl = pl.ds(j * 16, 16)
            idx_v[sl] = idx_v[sl] + _SCHUNK
            return carry2

        lax.fori_loop(0, _NVR, bump, 0, unroll=False)
        return carry

    lax.fori_loop(0, _SITERS, body, 0, unroll=False)
    plsc.subcore_barrier()
    pltpu.sync_copy(accum.at[pl.ds(t0, stripe)], agg2.at[c, pl.ds(t0, stripe)])


_sc_scatter = pl.kernel(
    _sc_scatter_body,
    out_type=jax.ShapeDtypeStruct((NC, N_NODES, DHP), jnp.float32),
    mesh=_mesh,
    scratch_types=[
        pltpu.VMEM((_SCHUNK,), jnp.int32),
        pltpu.VMEM((_SCHUNK,), jnp.int32),
        pltpu.VMEM((_SCHUNK, DHP), jnp.float32),
        pltpu.SemaphoreType.DMA,
        pltpu.VMEM_SHARED((N_NODES, DHP), jnp.float32),
    ],
    compiler_params=_sc_params,
)


_BNP = 2000


def _proj_body(nodes, ws, wr, p, q):
    n = nodes[...]
    p[...] = jnp.dot(n, ws[...], preferred_element_type=jnp.float32)
    q[...] = jnp.dot(n, wr[...], preferred_element_type=jnp.float32)


def _full2d(shape):
    return pl.BlockSpec(shape, lambda i: (0, 0))


_proj = pl.pallas_call(
    _proj_body,
    grid=(N_NODES // _BNP,),
    in_specs=[
        pl.BlockSpec((_BNP, DN), lambda i: (i, 0)),
        _full2d((DN, 64)),
        _full2d((DN, 64)),
    ],
    out_specs=[pl.BlockSpec((_BNP, 64), lambda i: (i, 0)),
               pl.BlockSpec((_BNP, 64), lambda i: (i, 0))],
    out_shape=[jax.ShapeDtypeStruct((N_NODES, 64), jnp.float32),
               jax.ShapeDtypeStruct((N_NODES, 64), jnp.float32)],
)


_BE = 6400
_BE8 = _BE // 8


def _edge_mlp_body(ef8, z8, w1, b1, w2, b2, w3, b3, out):
    h = jnp.maximum(
        jnp.dot(ef8[...], w1[...], preferred_element_type=jnp.float32)
        + z8[...] + b1[...], 0.0)
    h = jnp.maximum(
        jnp.dot(h, w2[...], preferred_element_type=jnp.float32) + b2[...], 0.0)
    y = jnp.dot(h, w3[...], preferred_element_type=jnp.float32) + b3[...]
    out[0] = y[:, :128]
    out[1] = y[:, 128:]


_edge_mlp = pl.pallas_call(
    _edge_mlp_body,
    grid=(N_EDGES // _BE,),
    in_specs=[
        pl.BlockSpec((_BE8, 128), lambda i: (i, 0)),
        pl.BlockSpec((_BE8, 512), lambda i: (i, 0)),
        _full2d((128, 512)),
        _full2d((1, 512)),
        _full2d((512, 256)),
        _full2d((1, 256)),
        _full2d((256, 256)),
        _full2d((1, 256)),
    ],
    out_specs=pl.BlockSpec((NC, _BE8, 128), lambda i: (0, i, 0)),
    out_shape=jax.ShapeDtypeStruct((NC, N_EDGES // 8, 128), jnp.float32),
)


_BN8 = 1600
_N8P = 12800


def _node_mlp_body(nodes8, agg8, w1, b1, wa, wb, w2, b2, w3, b3, out):
    h = (jnp.dot(nodes8[...], w1[...], preferred_element_type=jnp.float32)
         + jnp.dot(agg8[0], wa[...], preferred_element_type=jnp.float32)
         + jnp.dot(agg8[1], wb[...], preferred_element_type=jnp.float32)
         + b1[...])
    h = jnp.maximum(h, 0.0)
    h = jnp.maximum(
        jnp.dot(h, w2[...], preferred_element_type=jnp.float32) + b2[...], 0.0)
    out[...] = jnp.dot(h, w3[...], preferred_element_type=jnp.float32) + b3[...]


_node_mlp = pl.pallas_call(
    _node_mlp_body,
    grid=(_N8P // _BN8,),
    in_specs=[
        pl.BlockSpec((_BN8, 160), lambda i: (i, 0)),
        pl.BlockSpec((NC, _BN8, 128), lambda i: (0, i, 0)),
        _full2d((160, 512)),
        _full2d((1, 512)),
        _full2d((128, 512)),
        _full2d((128, 512)),
        _full2d((512, 256)),
        _full2d((1, 256)),
        _full2d((256, 160)),
        _full2d((1, 160)),
    ],
    out_specs=pl.BlockSpec((_BN8, 160), lambda i: (i, 0)),
    out_shape=jax.ShapeDtypeStruct((_N8P, 160), jnp.float32),
)


def _blockdiag(w, r):
    k, n = w.shape
    out = jnp.zeros((r * k, r * n), w.dtype)
    for i in range(r):
        out = lax.dynamic_update_slice(out, w, (i * k, i * n))
    return out


def kernel(nodes, edge_features, senders, receivers,
           ew1, eb1, ew2, eb2, ew3, eb3,
           nw1, nb1, nw2, nb2, nw3, nb3):
    p, q = _proj(nodes, ew1[DE:DE + DN], ew1[DE + DN:])
    z = _sc_gather(p, q, senders, receivers)
    ef8 = edge_features.reshape(N_EDGES // 8, 8 * DE)
    z8 = z.reshape(N_EDGES // 8, 8 * 64)
    w1e8 = _blockdiag(ew1[:DE], 8)
    b1_8 = (jnp.tile(eb1, 8)[None, :] + 0.0)
    w2_8 = _blockdiag(ew2, 8)
    b2_8 = jnp.tile(eb2, 8)[None, :]
    w3a = jnp.pad(ew3[:, :DH], ((0, 0), (0, DHP - DH)))
    w3b = jnp.pad(ew3[:, DH:], ((0, 0), (0, DHP - DH)))
    w3_8 = jnp.concatenate([_blockdiag(w3a, 8), _blockdiag(w3b, 8)], axis=1)
    b3a = jnp.pad(eb3[:DH], (0, DHP - DH))
    b3b = jnp.pad(eb3[DH:], (0, DHP - DH))
    b3_8 = jnp.concatenate([jnp.tile(b3a, 8), jnp.tile(b3b, 8)])[None, :]
    msgs = _edge_mlp(ef8, z8, w1e8, b1_8, w2_8, b2_8, w3_8, b3_8)
    agg2 = _sc_scatter(msgs.reshape(NC * N_EDGES, DHP), receivers,
                       jnp.zeros((N_NODES, DHP), jnp.float32))
    nodes8 = jnp.pad(nodes.reshape(N_NODES // 8, 8 * DN),
                     ((0, _N8P - N_NODES // 8), (0, 0)))
    agg8 = jnp.pad(agg2.reshape(NC, N_NODES // 8, 8 * DHP),
                   ((0, 0), (0, _N8P - N_NODES // 8), (0, 0)))
    w1n8 = _blockdiag(nw1[:DN], 8)
    w1a8 = _blockdiag(jnp.pad(nw1[DN:DN + DH], ((0, DHP - DH), (0, 0))), 8)
    w1b8 = _blockdiag(jnp.pad(nw1[DN + DH:], ((0, DHP - DH), (0, 0))), 8)
    nb1_8 = jnp.tile(nb1, 8)[None, :]
    w2n8 = _blockdiag(nw2, 8)
    nb2_8 = jnp.tile(nb2, 8)[None, :]
    w3n8 = _blockdiag(nw3, 8)
    nb3_8 = jnp.tile(nb3, 8)[None, :]
    out8 = _node_mlp(nodes8, agg8, w1n8, nb1_8, w1a8, w1b8,
                     w2n8, nb2_8, w3n8, nb3_8)
    return out8[:N_NODES // 8].reshape(N_NODES, DN)
